# baseline (speedup 1.0000x reference)
import functools

import jax
import jax.numpy as jnp
from jax import lax
from jax.experimental import pallas as pl
from jax.experimental.pallas import tpu as pltpu
from jax.experimental.pallas import tpu_sc as plsc

VOCAB = 1000000
D = 64
H = 128
B = 16384
L = 50

NC = 2
NS = 16
NW = NC * NS
PER_W = B // NW
CHUNK = 16
N_CHUNK = PER_W // CHUNK
ROWS = CHUNK * L
IDXCOLS = 100
IDXROWS = ROWS // IDXCOLS

NGRP = VOCAB // 128
VTAIL = VOCAB - NGRP * 128


def _sc_transpose(tT, tail_lin):
    mesh = plsc.VectorSubcoreMesh(core_axis_name="c", subcore_axis_name="s")

    @functools.partial(
        pl.kernel,
        out_type=jax.ShapeDtypeStruct((VOCAB * D,), jnp.float32),
        mesh=mesh,
        scratch_types=[
            pltpu.VMEM((D, 128), jnp.float32),
            pltpu.VMEM((128 * D,), jnp.float32),
            pltpu.VMEM((VTAIL * D,), jnp.float32),
        ],
        compiler_params=pltpu.CompilerParams(use_tc_tiling_on_sc=True,
                                             needs_layout_passes=False),
    )
    def k(tT_hbm, tail_hbm, out_hbm, blk_v, stage_v, tail_v):
        wid = lax.axis_index("s") * NC + lax.axis_index("c")
        lanes = jnp.arange(16, dtype=jnp.int32)
        kbase = [(16 * kk + lanes) * D for kk in range(8)]
        n_my = (NGRP - wid + NW - 1) // NW

        def grp_body(i, carry):
            g = pl.multiple_of(wid + i * NW, 1)
            pltpu.sync_copy(tT_hbm.at[:, pl.ds(g * 128, 128)], blk_v)

            def row_body(r, c2):
                for kk in range(8):
                    v = blk_v[r, pl.ds(16 * kk, 16)]
                    plsc.store_scatter(stage_v, [kbase[kk] + r], v)
                return c2

            lax.fori_loop(0, D, row_body, 0)
            pltpu.sync_copy(stage_v, out_hbm.at[pl.ds(g * 128 * D, 128 * D)])
            return carry

        lax.fori_loop(0, n_my, grp_body, 0)

        @pl.when(wid == 0)
        def _():
            pltpu.sync_copy(tail_hbm, tail_v)
            pltpu.sync_copy(tail_v, out_hbm.at[pl.ds(NGRP * 128 * D, VTAIL * D)])

    return k(tT, tail_lin)


def _sc_gather_sum(seq2d, item_flat, table_lin):
    mesh = plsc.VectorSubcoreMesh(core_axis_name="c", subcore_axis_name="s")

    @functools.partial(
        pl.kernel,
        out_type=(
            jax.ShapeDtypeStruct((B, D), jnp.float32),
            jax.ShapeDtypeStruct((B, D), jnp.float32),
        ),
        mesh=mesh,
        scratch_types=[
            pltpu.VMEM((IDXROWS, IDXCOLS), jnp.int32),
            pltpu.VMEM((ROWS, D), jnp.float32),
            pltpu.VMEM((CHUNK, D), jnp.float32),
            pltpu.VMEM((CHUNK,), jnp.int32),
            pltpu.VMEM((CHUNK, D), jnp.float32),
            pltpu.SemaphoreType.DMA,
            pltpu.SemaphoreType.DMA,
        ],
        compiler_params=pltpu.CompilerParams(use_tc_tiling_on_sc=False),
    )
    def k(seq_hbm, item_hbm, table_hbm, sum_hbm, item_out_hbm,
          idx_v, rows_v, stage_v, iidx_v, irows_v, sem, isem):
        wid = lax.axis_index("s") * NC + lax.axis_index("c")

        def chunk_body(c, carry):
            ebase = pl.multiple_of(wid * PER_W + c * CHUNK, CHUNK)
            irow = pl.multiple_of(ebase * L // IDXCOLS, 8)
            pltpu.sync_copy(seq_hbm.at[pl.ds(irow, IDXROWS)], idx_v)
            pltpu.sync_copy(item_hbm.at[pl.ds(ebase, CHUNK)], iidx_v)
            cps = []
            for j in range(IDXROWS):
                cps.append(pltpu.async_copy(
                    table_hbm.at[idx_v.at[j]],
                    rows_v.at[pl.ds(j * IDXCOLS, IDXCOLS)],
                    sem))
            icp = pltpu.async_copy(table_hbm.at[iidx_v], irows_v, isem)
            for cp in cps:
                cp.wait()
            for b in range(CHUNK):
                def l_body(l, acc):
                    r = b * L + l
                    return tuple(acc[d] + rows_v[r, pl.ds(d * 16, 16)]
                                 for d in range(4))
                acc = lax.fori_loop(
                    0, L, l_body,
                    tuple(jnp.zeros((16,), jnp.float32) for _ in range(4)))
                for d in range(4):
                    stage_v[b, pl.ds(d * 16, 16)] = acc[d]
            icp.wait()
            pltpu.sync_copy(stage_v, sum_hbm.at[pl.ds(ebase, CHUNK)])
            pltpu.sync_copy(irows_v, item_out_hbm.at[pl.ds(ebase, CHUNK)])
            return carry

        lax.fori_loop(0, N_CHUNK, chunk_body, 0)

    return k(seq2d, item_flat, table_lin)


BLK = 512


def _tc_mlp(sums, items, seq, t0, W1, b1, W2, b2):
    def body(sum_ref, item_ref, seq_ref, t0_ref, W1_ref, b1_ref, W2_ref,
             b2_ref, out_ref):
        idx = seq_ref[...]
        n0 = jnp.sum((idx == 0).astype(jnp.float32), axis=1, keepdims=True)
        mean = (sum_ref[...] - n0 * t0_ref[...]) * (1.0 / L)
        h = jnp.maximum(
            jnp.dot(mean, W1_ref[...], preferred_element_type=jnp.float32)
            + b1_ref[...], 0.0)
        u = jnp.maximum(
            jnp.dot(h, W2_ref[...], preferred_element_type=jnp.float32)
            + b2_ref[...], 0.0)
        logit = jnp.sum(u * item_ref[...], axis=1, keepdims=True)
        out_ref[...] = jax.nn.sigmoid(logit)

    return pl.pallas_call(
        body,
        grid=(B // BLK,),
        in_specs=[
            pl.BlockSpec((BLK, D), lambda i: (i, 0)),
            pl.BlockSpec((BLK, D), lambda i: (i, 0)),
            pl.BlockSpec((BLK, L), lambda i: (i, 0)),
            pl.BlockSpec((1, D), lambda i: (0, 0)),
            pl.BlockSpec((D, H), lambda i: (0, 0)),
            pl.BlockSpec((1, H), lambda i: (0, 0)),
            pl.BlockSpec((H, D), lambda i: (0, 0)),
            pl.BlockSpec((1, D), lambda i: (0, 0)),
        ],
        out_specs=pl.BlockSpec((BLK, 1), lambda i: (i, 0)),
        out_shape=jax.ShapeDtypeStruct((B, 1), jnp.float32),
    )(sums, items, seq, t0, W1, b1, W2, b2)


def kernel(seq_inputs, item_inputs, table, W1, b1, W2, b2):
    tail_lin = table[NGRP * 128:].reshape(VTAIL * D)
    table_flat = _sc_transpose(table.T, tail_lin)
    table_lin = table_flat.reshape(VOCAB, D)
    seq2d = seq_inputs.reshape(B * L // IDXCOLS, IDXCOLS)
    item_flat = item_inputs.reshape(B)
    sums, items = _sc_gather_sum(seq2d, item_flat, table_lin)
    t0 = table[0:1, :]
    return _tc_mlp(sums, items, seq_inputs, t0,
                   W1, b1.reshape(1, H), W2, b2.reshape(1, D))


# speedup vs baseline: 2.7222x; 2.7222x over previous
import functools

import jax
import jax.numpy as jnp
from jax import lax
from jax.experimental import pallas as pl
from jax.experimental.pallas import tpu as pltpu
from jax.experimental.pallas import tpu_sc as plsc

VOCAB = 1000000
D = 64
H = 128
B = 16384
L = 50

NC = 2
NS = 16
NW = NC * NS
PER_W = B // NW
CHUNK = 16
N_CHUNK = PER_W // CHUNK
ROWS = CHUNK * L
IDXCOLS = 100
IDXROWS = ROWS // IDXCOLS

NGRP = VOCAB // 128
VTAIL = VOCAB - NGRP * 128
GRP_MAIN = (NGRP // NW) & ~1
GRP_EXTRA = NGRP - GRP_MAIN * NW


def _sc_transpose(tT, tail_lin):
    mesh = plsc.VectorSubcoreMesh(core_axis_name="c", subcore_axis_name="s")

    @functools.partial(
        pl.kernel,
        out_type=jax.ShapeDtypeStruct((VOCAB * D,), jnp.float32),
        mesh=mesh,
        scratch_types=[
            pltpu.VMEM((D, 128), jnp.float32),
            pltpu.VMEM((D, 128), jnp.float32),
            pltpu.VMEM((128 * 65,), jnp.float32),
            pltpu.VMEM((128 * D,), jnp.float32),
            pltpu.VMEM((VTAIL * D,), jnp.float32),
            pltpu.SemaphoreType.DMA,
            pltpu.SemaphoreType.DMA,
        ],
        compiler_params=pltpu.CompilerParams(use_tc_tiling_on_sc=True,
                                             needs_layout_passes=False),
    )
    def k(tT_hbm, tail_hbm, out_hbm, blk0_v, blk1_v, stagea_v, stage_v,
          tail_v, sem0, sem1):
        wid = lax.axis_index("s") * NC + lax.axis_index("c")
        lanes = jnp.arange(16, dtype=jnp.int32)
        kbase = [(16 * kk + lanes) * 65 for kk in range(8)]

        def start(g, blk, sem):
            pltpu.async_copy(tT_hbm.at[:, pl.ds(g * 128, 128)], blk, sem)

        def wait(blk, sem):
            pltpu.make_async_copy(
                tT_hbm.at[:, pl.ds(0, 128)], blk, sem).wait()

        def emit_group(g, blk):
            def row8(r8, c2):
                for rr in range(8):
                    r = r8 * 8 + rr
                    vals = [blk[r, pl.ds(16 * kk, 16)] for kk in range(8)]
                    for kk in range(8):
                        plsc.store_scatter(stagea_v, [kbase[kk] + r],
                                           vals[kk])
                return c2

            lax.fori_loop(0, 8, row8, 0)

            def dens8(v8, c2):
                for vv in range(8):
                    v = v8 * 8 + vv
                    dvals = [stagea_v[pl.ds(v * 65 + 16 * dd, 16)]
                             for dd in range(4)]
                    for dd in range(4):
                        stage_v[pl.ds(v * 64 + 16 * dd, 16)] = dvals[dd]
                return c2

            lax.fori_loop(0, 16, dens8, 0)
            pltpu.sync_copy(stage_v, out_hbm.at[pl.ds(g * 128 * D, 128 * D)])

        start(wid, blk0_v, sem0)

        def body(i, carry):
            g0 = wid + (2 * i) * NW
            start(g0 + NW, blk1_v, sem1)
            wait(blk0_v, sem0)
            emit_group(g0, blk0_v)

            @pl.when(i < GRP_MAIN // 2 - 1)
            def _():
                start(g0 + 2 * NW, blk0_v, sem0)

            wait(blk1_v, sem1)
            emit_group(g0 + NW, blk1_v)
            return carry

        lax.fori_loop(0, GRP_MAIN // 2, body, 0)

        @pl.when(wid < GRP_EXTRA)
        def _():
            g = GRP_MAIN * NW + wid
            start(g, blk0_v, sem0)
            wait(blk0_v, sem0)
            emit_group(g, blk0_v)

        @pl.when(wid == GRP_EXTRA)
        def _():
            pltpu.sync_copy(tail_hbm, tail_v)
            pltpu.sync_copy(tail_v, out_hbm.at[pl.ds(NGRP * 128 * D, VTAIL * D)])

    return k(tT, tail_lin)


def _sc_gather_sum(seq2d, item_flat, table_lin):
    mesh = plsc.VectorSubcoreMesh(core_axis_name="c", subcore_axis_name="s")

    @functools.partial(
        pl.kernel,
        out_type=(
            jax.ShapeDtypeStruct((B, D), jnp.float32),
            jax.ShapeDtypeStruct((B, D), jnp.float32),
        ),
        mesh=mesh,
        scratch_types=[
            pltpu.VMEM((IDXROWS, IDXCOLS), jnp.int32),
            pltpu.VMEM((ROWS, D), jnp.float32),
            pltpu.VMEM((CHUNK, D), jnp.float32),
            pltpu.VMEM((CHUNK,), jnp.int32),
            pltpu.VMEM((CHUNK, D), jnp.float32),
            pltpu.SemaphoreType.DMA,
            pltpu.SemaphoreType.DMA,
        ],
        compiler_params=pltpu.CompilerParams(use_tc_tiling_on_sc=False),
    )
    def k(seq_hbm, item_hbm, table_hbm, sum_hbm, item_out_hbm,
          idx_v, rows_v, stage_v, iidx_v, irows_v, sem, isem):
        wid = lax.axis_index("s") * NC + lax.axis_index("c")

        def chunk_body(c, carry):
            ebase = pl.multiple_of(wid * PER_W + c * CHUNK, CHUNK)
            irow = pl.multiple_of(ebase * L // IDXCOLS, 8)
            pltpu.sync_copy(seq_hbm.at[pl.ds(irow, IDXROWS)], idx_v)
            pltpu.sync_copy(item_hbm.at[pl.ds(ebase, CHUNK)], iidx_v)
            cps = []
            for j in range(IDXROWS):
                cps.append(pltpu.async_copy(
                    table_hbm.at[idx_v.at[j]],
                    rows_v.at[pl.ds(j * IDXCOLS, IDXCOLS)],
                    sem))
            icp = pltpu.async_copy(table_hbm.at[iidx_v], irows_v, isem)
            for cp in cps:
                cp.wait()
            for b in range(CHUNK):
                def l_body(l, acc):
                    r = b * L + l
                    return tuple(acc[d] + rows_v[r, pl.ds(d * 16, 16)]
                                 for d in range(4))
                acc = lax.fori_loop(
                    0, L, l_body,
                    tuple(jnp.zeros((16,), jnp.float32) for _ in range(4)))
                for d in range(4):
                    stage_v[b, pl.ds(d * 16, 16)] = acc[d]
            icp.wait()
            pltpu.sync_copy(stage_v, sum_hbm.at[pl.ds(ebase, CHUNK)])
            pltpu.sync_copy(irows_v, item_out_hbm.at[pl.ds(ebase, CHUNK)])
            return carry

        lax.fori_loop(0, N_CHUNK, chunk_body, 0)

    return k(seq2d, item_flat, table_lin)


BLK = 512


def _tc_mlp(sums, items, seq, t0, W1, b1, W2, b2):
    def body(sum_ref, item_ref, seq_ref, t0_ref, W1_ref, b1_ref, W2_ref,
             b2_ref, out_ref):
        idx = seq_ref[...]
        n0 = jnp.sum((idx == 0).astype(jnp.float32), axis=1, keepdims=True)
        mean = (sum_ref[...] - n0 * t0_ref[...]) * (1.0 / L)
        h = jnp.maximum(
            jnp.dot(mean, W1_ref[...], preferred_element_type=jnp.float32)
            + b1_ref[...], 0.0)
        u = jnp.maximum(
            jnp.dot(h, W2_ref[...], preferred_element_type=jnp.float32)
            + b2_ref[...], 0.0)
        logit = jnp.sum(u * item_ref[...], axis=1, keepdims=True)
        out_ref[...] = jax.nn.sigmoid(logit)

    return pl.pallas_call(
        body,
        grid=(B // BLK,),
        in_specs=[
            pl.BlockSpec((BLK, D), lambda i: (i, 0)),
            pl.BlockSpec((BLK, D), lambda i: (i, 0)),
            pl.BlockSpec((BLK, L), lambda i: (i, 0)),
            pl.BlockSpec((1, D), lambda i: (0, 0)),
            pl.BlockSpec((D, H), lambda i: (0, 0)),
            pl.BlockSpec((1, H), lambda i: (0, 0)),
            pl.BlockSpec((H, D), lambda i: (0, 0)),
            pl.BlockSpec((1, D), lambda i: (0, 0)),
        ],
        out_specs=pl.BlockSpec((BLK, 1), lambda i: (i, 0)),
        out_shape=jax.ShapeDtypeStruct((B, 1), jnp.float32),
    )(sums, items, seq, t0, W1, b1, W2, b2)


def kernel(seq_inputs, item_inputs, table, W1, b1, W2, b2):
    tail_lin = table[NGRP * 128:].reshape(VTAIL * D)
    table_flat = _sc_transpose(table.T, tail_lin)
    table_lin = table_flat.reshape(VOCAB, D)
    seq2d = seq_inputs.reshape(B * L // IDXCOLS, IDXCOLS)
    item_flat = item_inputs.reshape(B)
    sums, items = _sc_gather_sum(seq2d, item_flat, table_lin)
    t0 = table[0:1, :]
    return _tc_mlp(sums, items, seq_inputs, t0,
                   W1, b1.reshape(1, H), W2, b2.reshape(1, D))


# speedup vs baseline: 3.2088x; 1.1787x over previous
import functools

import jax
import jax.numpy as jnp
from jax import lax
from jax.experimental import pallas as pl
from jax.experimental.pallas import tpu as pltpu
from jax.experimental.pallas import tpu_sc as plsc

VOCAB = 1000000
D = 64
H = 128
B = 16384
L = 50

NC = 2
NS = 16
NW = NC * NS
PER_W = B // NW
CHUNK = 16
N_CHUNK = PER_W // CHUNK
ROWS = CHUNK * L
IDXCOLS = 100
IDXROWS = ROWS // IDXCOLS

NGRP = VOCAB // 128
VTAIL = VOCAB - NGRP * 128
GRP_MAIN = (NGRP // NW) & ~1
GRP_EXTRA = NGRP - GRP_MAIN * NW


def _sc_transpose(tT, tail_lin):
    mesh = plsc.VectorSubcoreMesh(core_axis_name="c", subcore_axis_name="s")

    @functools.partial(
        pl.kernel,
        out_type=jax.ShapeDtypeStruct((VOCAB * D,), jnp.float32),
        mesh=mesh,
        scratch_types=[
            pltpu.VMEM((D, 128), jnp.float32),
            pltpu.VMEM((D, 128), jnp.float32),
            pltpu.VMEM((128 * 65,), jnp.float32),
            pltpu.VMEM((128 * D,), jnp.float32),
            pltpu.VMEM((VTAIL * D,), jnp.float32),
            pltpu.SemaphoreType.DMA,
            pltpu.SemaphoreType.DMA,
        ],
        compiler_params=pltpu.CompilerParams(use_tc_tiling_on_sc=True,
                                             needs_layout_passes=False),
    )
    def k(tT_hbm, tail_hbm, out_hbm, blk0_v, blk1_v, stagea_v, stage_v,
          tail_v, sem0, sem1):
        wid = lax.axis_index("s") * NC + lax.axis_index("c")
        lanes = jnp.arange(16, dtype=jnp.int32)
        kbase = [(16 * kk + lanes) * 65 for kk in range(8)]

        def start(g, blk, sem):
            pltpu.async_copy(tT_hbm.at[:, pl.ds(g * 128, 128)], blk, sem)

        def wait(blk, sem):
            pltpu.make_async_copy(
                tT_hbm.at[:, pl.ds(0, 128)], blk, sem).wait()

        def emit_group(g, blk):
            def row8(r8, c2):
                for rr in range(8):
                    r = r8 * 8 + rr
                    vals = [blk[r, pl.ds(16 * kk, 16)] for kk in range(8)]
                    for kk in range(8):
                        plsc.store_scatter(stagea_v, [kbase[kk] + r],
                                           vals[kk])
                return c2

            lax.fori_loop(0, 8, row8, 0)

            def dens8(v8, c2):
                for vv in range(8):
                    v = v8 * 8 + vv
                    dvals = [stagea_v[pl.ds(v * 65 + 16 * dd, 16)]
                             for dd in range(4)]
                    for dd in range(4):
                        stage_v[pl.ds(v * 64 + 16 * dd, 16)] = dvals[dd]
                return c2

            lax.fori_loop(0, 16, dens8, 0)
            pltpu.sync_copy(stage_v, out_hbm.at[pl.ds(g * 128 * D, 128 * D)])

        start(wid, blk0_v, sem0)

        def body(i, carry):
            g0 = wid + (2 * i) * NW
            start(g0 + NW, blk1_v, sem1)
            wait(blk0_v, sem0)
            emit_group(g0, blk0_v)

            @pl.when(i < GRP_MAIN // 2 - 1)
            def _():
                start(g0 + 2 * NW, blk0_v, sem0)

            wait(blk1_v, sem1)
            emit_group(g0 + NW, blk1_v)
            return carry

        lax.fori_loop(0, GRP_MAIN // 2, body, 0)

        @pl.when(wid < GRP_EXTRA)
        def _():
            g = GRP_MAIN * NW + wid
            start(g, blk0_v, sem0)
            wait(blk0_v, sem0)
            emit_group(g, blk0_v)

        @pl.when(wid == GRP_EXTRA)
        def _():
            pltpu.sync_copy(tail_hbm, tail_v)
            pltpu.sync_copy(tail_v, out_hbm.at[pl.ds(NGRP * 128 * D, VTAIL * D)])

    return k(tT, tail_lin)


def _sc_gather_sum(seq2d, item_flat, table_lin):
    mesh = plsc.VectorSubcoreMesh(core_axis_name="c", subcore_axis_name="s")

    @functools.partial(
        pl.kernel,
        out_type=(
            jax.ShapeDtypeStruct((B, D), jnp.float32),
            jax.ShapeDtypeStruct((B, D), jnp.float32),
        ),
        mesh=mesh,
        scratch_types=[
            pltpu.VMEM((IDXROWS, IDXCOLS), jnp.int32),
            pltpu.VMEM((IDXROWS, IDXCOLS), jnp.int32),
            pltpu.VMEM((ROWS, D), jnp.float32),
            pltpu.VMEM((ROWS, D), jnp.float32),
            pltpu.VMEM((CHUNK, D), jnp.float32),
            pltpu.VMEM((CHUNK,), jnp.int32),
            pltpu.VMEM((CHUNK, D), jnp.float32),
            pltpu.SemaphoreType.DMA,
            pltpu.SemaphoreType.DMA,
            pltpu.SemaphoreType.DMA,
        ],
        compiler_params=pltpu.CompilerParams(use_tc_tiling_on_sc=False),
    )
    def k(seq_hbm, item_hbm, table_hbm, sum_hbm, item_out_hbm,
          idx0_v, idx1_v, rows0_v, rows1_v, stage_v, iidx_v, irows_v,
          sem0, sem1, isem):
        wid = lax.axis_index("s") * NC + lax.axis_index("c")

        def load_idx(c, idx_v):
            ebase = pl.multiple_of(wid * PER_W + c * CHUNK, CHUNK)
            irow = pl.multiple_of(ebase * L // IDXCOLS, 8)
            pltpu.sync_copy(seq_hbm.at[pl.ds(irow, IDXROWS)], idx_v)

        def start_gathers(idx_v, rows_v, sem):
            for j in range(IDXROWS):
                pltpu.async_copy(
                    table_hbm.at[idx_v.at[j]],
                    rows_v.at[pl.ds(j * IDXCOLS, IDXCOLS)],
                    sem)

        def drain(rows_v, sem):
            pltpu.make_async_copy(
                table_hbm.at[pl.ds(0, ROWS)], rows_v, sem).wait()

        def consume(c, rows_v):
            ebase = pl.multiple_of(wid * PER_W + c * CHUNK, CHUNK)
            pltpu.sync_copy(item_hbm.at[pl.ds(ebase, CHUNK)], iidx_v)
            icp = pltpu.async_copy(table_hbm.at[iidx_v], irows_v, isem)
            for b in range(CHUNK):
                def l_body(l, acc):
                    r = b * L + 2 * l
                    acc = tuple(acc[d] + rows_v[r, pl.ds(d * 16, 16)]
                                for d in range(4))
                    return tuple(acc[d] + rows_v[r + 1, pl.ds(d * 16, 16)]
                                 for d in range(4))
                acc = lax.fori_loop(
                    0, L // 2, l_body,
                    tuple(jnp.zeros((16,), jnp.float32) for _ in range(4)))
                for d in range(4):
                    stage_v[b, pl.ds(d * 16, 16)] = acc[d]
            icp.wait()
            pltpu.sync_copy(stage_v, sum_hbm.at[pl.ds(ebase, CHUNK)])
            pltpu.sync_copy(irows_v, item_out_hbm.at[pl.ds(ebase, CHUNK)])

        load_idx(0, idx0_v)
        start_gathers(idx0_v, rows0_v, sem0)

        def pair_body(i, carry):
            c0 = 2 * i
            load_idx(c0 + 1, idx1_v)
            start_gathers(idx1_v, rows1_v, sem1)
            drain(rows0_v, sem0)
            consume(c0, rows0_v)

            @pl.when(i < N_CHUNK // 2 - 1)
            def _():
                load_idx(c0 + 2, idx0_v)
                start_gathers(idx0_v, rows0_v, sem0)

            drain(rows1_v, sem1)
            consume(c0 + 1, rows1_v)
            return carry

        lax.fori_loop(0, N_CHUNK // 2, pair_body, 0)

    return k(seq2d, item_flat, table_lin)


BLK = 512


def _tc_mlp(sums, items, seq, t0, W1, b1, W2, b2):
    def body(sum_ref, item_ref, seq_ref, t0_ref, W1_ref, b1_ref, W2_ref,
             b2_ref, out_ref):
        idx = seq_ref[...]
        n0 = jnp.sum((idx == 0).astype(jnp.float32), axis=1, keepdims=True)
        mean = (sum_ref[...] - n0 * t0_ref[...]) * (1.0 / L)
        h = jnp.maximum(
            jnp.dot(mean, W1_ref[...], preferred_element_type=jnp.float32)
            + b1_ref[...], 0.0)
        u = jnp.maximum(
            jnp.dot(h, W2_ref[...], preferred_element_type=jnp.float32)
            + b2_ref[...], 0.0)
        logit = jnp.sum(u * item_ref[...], axis=1, keepdims=True)
        out_ref[...] = jax.nn.sigmoid(logit)

    return pl.pallas_call(
        body,
        grid=(B // BLK,),
        in_specs=[
            pl.BlockSpec((BLK, D), lambda i: (i, 0)),
            pl.BlockSpec((BLK, D), lambda i: (i, 0)),
            pl.BlockSpec((BLK, L), lambda i: (i, 0)),
            pl.BlockSpec((1, D), lambda i: (0, 0)),
            pl.BlockSpec((D, H), lambda i: (0, 0)),
            pl.BlockSpec((1, H), lambda i: (0, 0)),
            pl.BlockSpec((H, D), lambda i: (0, 0)),
            pl.BlockSpec((1, D), lambda i: (0, 0)),
        ],
        out_specs=pl.BlockSpec((BLK, 1), lambda i: (i, 0)),
        out_shape=jax.ShapeDtypeStruct((B, 1), jnp.float32),
    )(sums, items, seq, t0, W1, b1, W2, b2)


def kernel(seq_inputs, item_inputs, table, W1, b1, W2, b2):
    tail_lin = table[NGRP * 128:].reshape(VTAIL * D)
    table_flat = _sc_transpose(table.T, tail_lin)
    table_lin = table_flat.reshape(VOCAB, D)
    seq2d = seq_inputs.reshape(B * L // IDXCOLS, IDXCOLS)
    item_flat = item_inputs.reshape(B)
    sums, items = _sc_gather_sum(seq2d, item_flat, table_lin)
    t0 = table[0:1, :]
    return _tc_mlp(sums, items, seq_inputs, t0,
                   W1, b1.reshape(1, H), W2, b2.reshape(1, D))
